# Initial kernel scaffold; baseline (speedup 1.0000x reference)
import jax
import jax.numpy as jnp
from jax.experimental import pallas as pl


def kernel(x, table):
    raise NotImplementedError("write your pallas kernel here")


# speedup vs baseline: 1.5656x; 1.5656x over previous
import functools

import jax
import jax.numpy as jnp
from jax import lax
from jax.experimental import pallas as pl
from jax.experimental.pallas import tpu as pltpu
from jax.experimental.pallas import tpu_sc as plsc

B = 4096
L = 200
D = 32
NBINS = 256
NC, NS = 2, 16
NW = NC * NS
SPAN = (L * B) // NW
CPW = 128
NCH = SPAN // CPW


def _prep_body(x_ref, si_ref, ls_ref, bs_ref, idx_ref):
    x = x_ref[...]
    tio = lax.broadcasted_iota(jnp.int32, (B, L), 1)
    fz = jnp.min(jnp.where(x == 0, tio, L), axis=1, keepdims=True)
    lens = jnp.where((fz == 0) | (fz == L), L, fz)

    bins = lax.broadcasted_iota(jnp.int32, (B, NBINS), 1)
    oh = (lens == bins).astype(jnp.float32)
    cnt = jnp.sum(oh, axis=0, keepdims=True)

    del cnt
    a_io = lax.broadcasted_iota(jnp.int32, (NBINS, NBINS), 0)
    b_io = lax.broadcasted_iota(jnp.int32, (NBINS, NBINS), 1)
    gt = (a_io > b_io).astype(jnp.float32)
    ohgt = jnp.dot(oh, gt, preferred_element_type=jnp.float32)
    cnt_gt = jnp.sum(ohgt, axis=0, keepdims=True)
    bs_ref[...] = cnt_gt.astype(jnp.int32)

    gr = jnp.sum(oh * cnt_gt, axis=1, keepdims=True)
    CH = 512
    r_io = lax.broadcasted_iota(jnp.int32, (CH, CH), 0)
    c_io = lax.broadcasted_iota(jnp.int32, (CH, CH), 1)
    ltri = (r_io >= c_io).astype(jnp.float32)
    carry = jnp.zeros((1, NBINS), jnp.float32)
    rank_parts = []
    for c in range(B // CH):
        ohc = lax.slice(oh, (c * CH, 0), ((c + 1) * CH, NBINS))
        incl = jnp.dot(ltri, ohc, preferred_element_type=jnp.float32) + carry
        ec = jnp.sum(incl * ohc, axis=1, keepdims=True) - 1.0
        grc = lax.slice(gr, (c * CH, 0), ((c + 1) * CH, 1))
        rank_parts.append(grc + ec)
        carry = carry + jnp.sum(ohc, axis=0, keepdims=True)
    rank = jnp.concatenate(rank_parts, axis=0).astype(jnp.int32)

    bvec = lax.broadcasted_iota(jnp.int32, (1, B), 1)
    m3 = jnp.concatenate([
        (bvec & 0xFF).astype(jnp.float32),
        (bvec >> 8).astype(jnp.float32),
        jnp.reshape(lens.astype(jnp.float32), (1, B)),
    ], axis=0)
    x_b0 = (x & 0xFF).astype(jnp.float32)
    x_b1 = ((x >> 8) & 0xFF).astype(jnp.float32)
    x_b2 = ((x >> 16) & 0xFF).astype(jnp.float32)
    dnum = (((0,), (0,)), ((), ()))
    RC = 512
    for c in range(B // RC):
        rbins = lax.broadcasted_iota(jnp.int32, (B, RC), 1) + (c * RC)
        ohr = (rank == rbins).astype(jnp.float32)
        res = jnp.dot(m3, ohr, preferred_element_type=jnp.float32)
        si_c = (lax.slice(res, (0, 0), (1, RC)).astype(jnp.int32)
                + (lax.slice(res, (1, 0), (2, RC)).astype(jnp.int32) << 8))
        si_ref[:, c * RC:(c + 1) * RC] = si_c
        ls_ref[:, c * RC:(c + 1) * RC] = lax.slice(res, (2, 0), (3, RC)).astype(jnp.int32)

        b0_c = lax.dot_general(ohr, x_b0, dnum,
                               preferred_element_type=jnp.float32)
        b1_c = lax.dot_general(ohr, x_b1, dnum,
                               preferred_element_type=jnp.float32)
        b2_c = lax.dot_general(ohr, x_b2, dnum,
                               preferred_element_type=jnp.float32)
        xs = (b0_c.astype(jnp.int32) + (b1_c.astype(jnp.int32) << 8)
              + (b2_c.astype(jnp.int32) << 16))
        lscol = jnp.reshape(lax.slice(res, (2, 0), (3, RC)), (RC, 1)).astype(jnp.int32)
        tio2 = lax.broadcasted_iota(jnp.int32, (RC, L), 1)
        xs_m = jnp.where(tio2 < lscol, xs, 0)
        idx_ref[:, c * RC:(c + 1) * RC] = xs_m.T


_prep = pl.pallas_call(
    _prep_body,
    out_shape=[
        jax.ShapeDtypeStruct((1, B), jnp.int32),
        jax.ShapeDtypeStruct((1, B), jnp.int32),
        jax.ShapeDtypeStruct((1, NBINS), jnp.int32),
        jax.ShapeDtypeStruct((L, B), jnp.int32),
    ],
)


def _sc_body(idx_hbm, table_hbm, out_hbm, idx_v, rows_v, sem_g):
    wid = lax.axis_index("s") * NC + lax.axis_index("c")
    span0 = wid * SPAN
    iota16 = lax.broadcasted_iota(jnp.int32, (16,), 0)
    zeros16 = jnp.zeros((16,), jnp.float32)

    pltpu.sync_copy(idx_hbm.at[pl.ds(span0, SPAN)], idx_v)

    def step(j, _):
        jb = j * CPW
        pltpu.async_copy(table_hbm.at[idx_v.at[pl.ds(jb, CPW)]],
                         rows_v, sem_g).wait()

        bad = jnp.bool_(False)
        for k in range(8):
            xm = plsc.load_gather(idx_v, [jb + iota16 + 16 * k])
            bad = bad | jnp.any(xm == 0)

        @pl.when(bad)
        def _fix():
            def fk(k2, _):
                rows16 = iota16 + 16 * k2
                xm = plsc.load_gather(idx_v, [jb + rows16])
                m = xm == 0

                def fc(cj, _):
                    cvec = jnp.full((16,), cj, jnp.int32)
                    plsc.store_scatter(rows_v, [rows16, cvec], zeros16, mask=m)
                    return 0
                return lax.fori_loop(0, D, fc, 0)
            lax.fori_loop(0, 8, fk, 0)

        pltpu.sync_copy(rows_v, out_hbm.at[pl.ds(span0 + jb, CPW)])
        return 0

    lax.fori_loop(0, NCH, step, 0)


@functools.cache
def _make_sc_main():
    return functools.partial(
        pl.kernel,
        out_type=jax.ShapeDtypeStruct((L * B, D), jnp.float32),
        mesh=plsc.VectorSubcoreMesh(core_axis_name="c", subcore_axis_name="s",
                                    num_cores=NC, num_subcores=NS),
        compiler_params=pltpu.CompilerParams(use_tc_tiling_on_sc=False,
                                             needs_layout_passes=False),
        scratch_types=[
            pltpu.VMEM((SPAN,), jnp.int32),
            pltpu.VMEM((CPW, D), jnp.float32),
            pltpu.SemaphoreType.DMA,
        ],
    )(_sc_body)


def kernel(x, table):
    si2, ls2, bs2, idx2 = _prep(x)
    si = si2.reshape(B)
    idx_flat = idx2.reshape(L * B)
    packed = _make_sc_main()(idx_flat, table)
    batch_sizes = bs2.reshape(NBINS)[:L]
    return packed, batch_sizes, si


# speedup vs baseline: 1.8027x; 1.1514x over previous
import functools

import jax
import jax.numpy as jnp
from jax import lax
from jax.experimental import pallas as pl
from jax.experimental.pallas import tpu as pltpu
from jax.experimental.pallas import tpu_sc as plsc

B = 4096
L = 200
D = 32
NBINS = 256
NC, NS = 2, 16
NW = NC * NS
SPAN = (L * B) // NW
CPW = 128
NCH = SPAN // CPW


def _prep_body(x_ref, si_ref, ls_ref, bs_ref, idx_ref):
    x = x_ref[...]
    tio = lax.broadcasted_iota(jnp.int32, (B, L), 1)
    fz = jnp.min(jnp.where(x == 0, tio, L), axis=1, keepdims=True)
    lens = jnp.where((fz == 0) | (fz == L), L, fz)

    bins = lax.broadcasted_iota(jnp.int32, (B, NBINS), 1)
    oh = (lens == bins).astype(jnp.float32)
    cnt = jnp.sum(oh, axis=0, keepdims=True)

    del cnt
    a_io = lax.broadcasted_iota(jnp.int32, (NBINS, NBINS), 0)
    b_io = lax.broadcasted_iota(jnp.int32, (NBINS, NBINS), 1)
    gt = (a_io > b_io).astype(jnp.float32)
    ohgt = jnp.dot(oh, gt, preferred_element_type=jnp.float32)
    cnt_gt = jnp.sum(ohgt, axis=0, keepdims=True)
    bs_ref[...] = cnt_gt.astype(jnp.int32)

    gr = jnp.sum(oh * cnt_gt, axis=1, keepdims=True)
    CH = 512
    r_io = lax.broadcasted_iota(jnp.int32, (CH, CH), 0)
    c_io = lax.broadcasted_iota(jnp.int32, (CH, CH), 1)
    ltri = (r_io >= c_io).astype(jnp.float32)
    carry = jnp.zeros((1, NBINS), jnp.float32)
    rank_parts = []
    for c in range(B // CH):
        ohc = lax.slice(oh, (c * CH, 0), ((c + 1) * CH, NBINS))
        incl = jnp.dot(ltri, ohc, preferred_element_type=jnp.float32) + carry
        ec = jnp.sum(incl * ohc, axis=1, keepdims=True) - 1.0
        grc = lax.slice(gr, (c * CH, 0), ((c + 1) * CH, 1))
        rank_parts.append(grc + ec)
        carry = carry + jnp.sum(ohc, axis=0, keepdims=True)
    rank = jnp.concatenate(rank_parts, axis=0).astype(jnp.int32)

    bvec = lax.broadcasted_iota(jnp.int32, (1, B), 1)
    m3 = jnp.concatenate([
        (bvec & 0xFF).astype(jnp.float32),
        (bvec >> 8).astype(jnp.float32),
        jnp.reshape(lens.astype(jnp.float32), (1, B)),
    ], axis=0)
    x_b0 = (x & 0xFF).astype(jnp.float32)
    x_b1 = ((x >> 8) & 0xFF).astype(jnp.float32)
    x_b2 = ((x >> 16) & 0xFF).astype(jnp.float32)
    dnum = (((0,), (0,)), ((), ()))
    RC = 512
    for c in range(B // RC):
        rbins = lax.broadcasted_iota(jnp.int32, (B, RC), 1) + (c * RC)
        ohr = (rank == rbins).astype(jnp.float32)
        res = jnp.dot(m3, ohr, preferred_element_type=jnp.float32)
        si_c = (lax.slice(res, (0, 0), (1, RC)).astype(jnp.int32)
                + (lax.slice(res, (1, 0), (2, RC)).astype(jnp.int32) << 8))
        si_ref[:, c * RC:(c + 1) * RC] = si_c
        ls_ref[:, c * RC:(c + 1) * RC] = lax.slice(res, (2, 0), (3, RC)).astype(jnp.int32)

        b0_c = lax.dot_general(ohr, x_b0, dnum,
                               preferred_element_type=jnp.float32)
        b1_c = lax.dot_general(ohr, x_b1, dnum,
                               preferred_element_type=jnp.float32)
        b2_c = lax.dot_general(ohr, x_b2, dnum,
                               preferred_element_type=jnp.float32)
        xs = (b0_c.astype(jnp.int32) + (b1_c.astype(jnp.int32) << 8)
              + (b2_c.astype(jnp.int32) << 16))
        lscol = jnp.reshape(lax.slice(res, (2, 0), (3, RC)), (RC, 1)).astype(jnp.int32)
        tio2 = lax.broadcasted_iota(jnp.int32, (RC, L), 1)
        xs_m = jnp.where(tio2 < lscol, xs, 0)
        idx_ref[:, c * RC:(c + 1) * RC] = xs_m.T


_prep = pl.pallas_call(
    _prep_body,
    out_shape=[
        jax.ShapeDtypeStruct((1, B), jnp.int32),
        jax.ShapeDtypeStruct((1, B), jnp.int32),
        jax.ShapeDtypeStruct((1, NBINS), jnp.int32),
        jax.ShapeDtypeStruct((L, B), jnp.int32),
    ],
)


GB = 10
GROUPS = NCH // GB
GROWS = GB * CPW


def _sc_body(idx_hbm, table_hbm, out_hbm, idx_v, rows_v, sem_g, sem_o):
    wid = lax.axis_index("s") * NC + lax.axis_index("c")
    span0 = wid * SPAN
    iota16 = lax.broadcasted_iota(jnp.int32, (16,), 0)
    zeros16 = jnp.zeros((16,), jnp.float32)

    pltpu.sync_copy(idx_hbm.at[pl.ds(span0, SPAN)], idx_v)

    def superstep(s, _):
        for d in (0, 1):
            g = 2 * s + d
            gb = g * GROWS

            @pl.when(s > 0)
            def _drain():
                pltpu.make_async_copy(rows_v.at[d],
                                      out_hbm.at[pl.ds(0, GROWS)],
                                      sem_o).wait()

            for b in range(GB):
                jb = gb + b * CPW
                pltpu.async_copy(
                    table_hbm.at[idx_v.at[pl.ds(jb, CPW)]],
                    rows_v.at[d, pl.ds(b * CPW, CPW)], sem_g)

            bads = []
            for b in range(GB):
                jb = gb + b * CPW
                bad = jnp.bool_(False)
                for k in range(8):
                    xm = plsc.load_gather(idx_v, [jb + iota16 + 16 * k])
                    bad = bad | jnp.any(xm == 0)
                bads.append(bad)

            for b in range(GB):
                jb = gb + b * CPW
                pltpu.make_async_copy(
                    table_hbm.at[idx_v.at[pl.ds(jb, CPW)]],
                    rows_v.at[d, pl.ds(b * CPW, CPW)], sem_g).wait()

                @pl.when(bads[b])
                def _fix(jb=jb, b=b):
                    def fk(k2, _):
                        rows16 = iota16 + 16 * k2
                        xm = plsc.load_gather(idx_v, [jb + rows16])
                        m = xm == 0

                        def fc(cj, _):
                            cvec = jnp.full((16,), cj, jnp.int32)
                            plsc.store_scatter(
                                rows_v, [jnp.full((16,), d, jnp.int32),
                                         b * CPW + rows16, cvec],
                                zeros16, mask=m)
                            return 0
                        return lax.fori_loop(0, D, fc, 0)
                    lax.fori_loop(0, 8, fk, 0)

            pltpu.async_copy(rows_v.at[d],
                             out_hbm.at[pl.ds(span0 + gb, GROWS)], sem_o)
        return 0

    lax.fori_loop(0, GROUPS // 2, superstep, 0)
    for _ in range(2):
        pltpu.make_async_copy(rows_v.at[0], out_hbm.at[pl.ds(0, GROWS)],
                              sem_o).wait()


@functools.cache
def _make_sc_main():
    return functools.partial(
        pl.kernel,
        out_type=jax.ShapeDtypeStruct((L * B, D), jnp.float32),
        mesh=plsc.VectorSubcoreMesh(core_axis_name="c", subcore_axis_name="s",
                                    num_cores=NC, num_subcores=NS),
        compiler_params=pltpu.CompilerParams(use_tc_tiling_on_sc=False,
                                             needs_layout_passes=False),
        scratch_types=[
            pltpu.VMEM((SPAN,), jnp.int32),
            pltpu.VMEM((2, GROWS, D), jnp.float32),
            pltpu.SemaphoreType.DMA,
            pltpu.SemaphoreType.DMA,
        ],
    )(_sc_body)


def kernel(x, table):
    si2, ls2, bs2, idx2 = _prep(x)
    si = si2.reshape(B)
    idx_flat = idx2.reshape(L * B)
    packed = _make_sc_main()(idx_flat, table)
    batch_sizes = bs2.reshape(NBINS)[:L]
    return packed, batch_sizes, si
